# baseline (speedup 1.0000x reference)
import functools

import jax
import jax.numpy as jnp
from jax import lax
from jax.experimental import pallas as pl
from jax.experimental.pallas import tpu as pltpu
from jax.experimental.pallas import tpu_sc as plsc

_NC = 2
_NS = 16
_NW = _NC * _NS
_L = 16
_D = 32


@functools.cache
def _build(batch: int):
    assert batch % (_NW * _L) == 0
    bpw = batch // _NW
    chunks = bpw // _L

    mesh = plsc.VectorSubcoreMesh(core_axis_name="c", subcore_axis_name="s")

    @functools.partial(
        pl.kernel,
        mesh=mesh,
        out_type=jax.ShapeDtypeStruct((batch,), jnp.float32),
        compiler_params=pltpu.CompilerParams(
            use_tc_tiling_on_sc=False, needs_layout_passes=False),
        scratch_types=[
            pltpu.VMEM((bpw,), jnp.int32),
            pltpu.VMEM((bpw,), jnp.int32),
            pltpu.VMEM((bpw, _D // 2), jnp.int32),
            pltpu.VMEM((bpw, _D // 2), jnp.int32),
            pltpu.VMEM((3 * _L,), jnp.float32),
            pltpu.VMEM((bpw,), jnp.float32),
            pltpu.SemaphoreType.DMA,
            pltpu.SemaphoreType.DMA,
        ],
    )
    def gmf(uidx_hbm, iidx_hbm, ut_hbm, it_hbm, wb_hbm, out_hbm,
            uidx_v, iidx_v, urows_v, irows_v, wb_v, out_v, sem_u, sem_i):
        wid = lax.axis_index("s") * _NC + lax.axis_index("c")
        base = wid * bpw

        pltpu.sync_copy(uidx_hbm.at[pl.ds(base, bpw)], uidx_v)
        pltpu.sync_copy(iidx_hbm.at[pl.ds(base, bpw)], iidx_v)
        cu = pltpu.async_copy(ut_hbm.at[uidx_v], urows_v, sem_u)
        ci = pltpu.async_copy(it_hbm.at[iidx_v], irows_v, sem_i)
        pltpu.sync_copy(wb_hbm, wb_v)
        cu.wait()
        ci.wait()

        w0 = wb_v[pl.ds(0, _L)]
        w1 = wb_v[pl.ds(_L, _L)]
        bv = wb_v[pl.ds(2 * _L, _L)]
        lane = lax.broadcasted_iota(jnp.int32, (_L,), 0)
        perms = [lane ^ (1 << k) for k in range(4)]

        def chunk_body(c, carry):
            r0 = c * _L
            acc = jnp.zeros((_L,), jnp.float32)
            for j in range(_L):
                r = r0 + j
                ub = plsc.bitcast(urows_v[r, pl.ds(0, _D // 2)], jnp.bfloat16)
                ib = plsc.bitcast(irows_v[r, pl.ds(0, _D // 2)], jnp.bfloat16)
                u0, u1 = plsc.unpack(ub, format=plsc.PackFormat.INTERLEAVED)
                i0, i1 = plsc.unpack(ib, format=plsc.PackFormat.INTERLEAVED)
                p = u0 * i0 * w0 + u1 * i1 * w1
                for pm in perms:
                    p = p + p.at[pm].get(mode="promise_in_bounds")
                acc = jnp.where(lane == j, p, acc)
            y = 1.0 / (1.0 + jnp.exp(-(acc + bv)))
            out_v[pl.ds(r0, _L)] = y
            return carry

        lax.fori_loop(0, chunks, chunk_body, 0)
        pltpu.sync_copy(out_v, out_hbm.at[pl.ds(base, bpw)])

    return gmf


def kernel(user_indices, item_indices, user_table, item_table, W, b):
    batch = user_indices.shape[0]
    uidx = user_indices.astype(jnp.int32)
    iidx = item_indices.astype(jnp.int32)
    wf = W.reshape(-1).astype(jnp.float32)
    wb = jnp.concatenate([
        wf[0::2],
        wf[1::2],
        jnp.broadcast_to(b.reshape(-1).astype(jnp.float32), (_L,)),
    ])
    ut32 = lax.bitcast_convert_type(
        user_table.astype(jnp.bfloat16).reshape(-1, _D // 2, 2), jnp.int32)
    it32 = lax.bitcast_convert_type(
        item_table.astype(jnp.bfloat16).reshape(-1, _D // 2, 2), jnp.int32)
    out = _build(batch)(uidx, iidx, ut32, it32, wb)
    return out.reshape(-1, 1)


# speedup vs baseline: 5.0465x; 5.0465x over previous
import functools

import jax
import jax.numpy as jnp
from jax import lax
from jax.experimental import pallas as pl
from jax.experimental.pallas import tpu as pltpu
from jax.experimental.pallas import tpu_sc as plsc

_NC = 2
_NS = 16
_NW = _NC * _NS
_L = 16
_D = 32
_B = 16384
_CW = 1024
_SCR_ROWS = _B + 64
_SINK = _B

_CP = pltpu.CompilerParams(use_tc_tiling_on_sc=False, needs_layout_passes=False)
_CPT = pltpu.CompilerParams(use_tc_tiling_on_sc=True, needs_layout_passes=False)

_LANE = None


def _scan_bin(idx_v, hits_v, wid, nsteps, tbits):
    lane = lax.broadcasted_iota(jnp.int32, (_L,), 0)

    def step(j, ptr):
        v = idx_v[pl.ds(j * _L, _L)]
        cid = lax.shift_right_logical(v, 10)
        mine = (cid & 31) == wid
        m32 = jnp.where(mine, 1, 0)
        ranks = plsc.cumsum(m32)
        tot = ranks[_L - 1]
        k = j * _L + lane
        t = lax.shift_right_logical(cid, 5)
        packed = lax.shift_left(t, 25) | lax.shift_left(k, 10) | (v & 1023)
        plsc.store_scatter(hits_v, [ptr + ranks - 1], packed, mask=mine)
        return ptr + tot

    return lax.fori_loop(0, nsteps, step, 0)


def _rescan(hits_v, n, t, clist_v):
    nsteps = (n + _L - 1) // _L
    lane = lax.broadcasted_iota(jnp.int32, (_L,), 0)

    def step(j, ptr):
        v = hits_v[pl.ds(j * _L, _L)]
        valid = (j * _L + lane) < n
        mine = jnp.logical_and(valid, lax.shift_right_logical(v, 25) == t)
        m32 = jnp.where(mine, 1, 0)
        ranks = plsc.cumsum(m32)
        tot = ranks[_L - 1]
        plsc.store_scatter(clist_v, [ptr + ranks - 1], v, mask=mine)
        return ptr + tot

    return lax.while_loop(
        lambda c: c[0] < nsteps,
        lambda c: (c[0] + 1, step(c[0], c[1])),
        (0, 0),
    )[1]


def _group_ids(clist_v, g, m):
    lane = lax.broadcasted_iota(jnp.int32, (_L,), 0)
    v = clist_v[pl.ds(g, _L)]
    valid = (g + lane) < m
    k = jnp.where(valid, lax.shift_right_logical(v, 10) & 0x7FFF,
                  _SINK + lane)
    off = jnp.where(valid, v & 1023, 0)
    return k, off


@functools.cache
def _build_item():
    mesh = plsc.VectorSubcoreMesh(core_axis_name="c", subcore_axis_name="s")
    nchunks = 98
    slots = 4

    @functools.partial(
        pl.kernel,
        mesh=mesh,
        out_type=jax.ShapeDtypeStruct((_SCR_ROWS, 128), jnp.float32),
        compiler_params=_CPT,
        scratch_types=[
            pltpu.VMEM((_B,), jnp.int32),
            pltpu.VMEM((_B,), jnp.int32),
            pltpu.VMEM((_B,), jnp.int32),
            pltpu.VMEM((_D, _CW), jnp.float32),
            pltpu.VMEM((_L, 129), jnp.float32),
            pltpu.VMEM((_L, 128), jnp.float32),
            pltpu.VMEM((_L,), jnp.int32),
            pltpu.SemaphoreType.DMA,
        ],
    )
    def kern(iidx_hbm, it_hbm, iscr_hbm,
             idx_v, hits_v, clist_v, chunk_v, gp_v, gc_v, kg_v, sem):
        wid = lax.axis_index("s") * _NC + lax.axis_index("c")
        pltpu.sync_copy(iidx_hbm, idx_v)
        n = _scan_bin(idx_v, hits_v, wid, _B // _L, 25)
        lane = lax.broadcasted_iota(jnp.int32, (_L,), 0)

        def slot_body(t, carry):
            c = wid + 32 * t

            @pl.when(c < nchunks)
            def _():
                base = pl.multiple_of(c * _CW, 128)

                @pl.when(c < nchunks - 1)
                def _():
                    pltpu.sync_copy(it_hbm.at[:, pl.ds(base, _CW)], chunk_v)

                @pl.when(c == nchunks - 1)
                def _():
                    pltpu.sync_copy(it_hbm.at[:, pl.ds(base, 768)],
                                    chunk_v.at[:, pl.ds(0, 768)])

                m = _rescan(hits_v, n, t, clist_v)

                def grp_body(g):
                    k, off = _group_ids(clist_v, g, m)
                    kg_v[pl.ds(0, _L)] = k
                    for d in range(_D):
                        dv = jnp.full((_L,), d, jnp.int32)
                        val = plsc.load_gather(chunk_v, [dv, off])
                        plsc.store_scatter(gp_v, [lane, dv], val)
                    pltpu.async_copy(gp_v.at[:, pl.ds(0, 128)],
                                     iscr_hbm.at[kg_v], sem).wait()
                    return g + _L

                lax.while_loop(lambda g: g < m, grp_body, 0)

            return carry

        lax.fori_loop(0, slots, slot_body, 0)
        return None

    return kern


@functools.cache
def _build_user():
    mesh = plsc.VectorSubcoreMesh(core_axis_name="c", subcore_axis_name="s")
    nchunks = 977
    slots = 31

    @functools.partial(
        pl.kernel,
        mesh=mesh,
        out_type=jax.ShapeDtypeStruct((_SCR_ROWS, 128), jnp.float32),
        compiler_params=_CPT,
        scratch_types=[
            pltpu.VMEM((_B,), jnp.int32),
            pltpu.VMEM((_B,), jnp.int32),
            pltpu.VMEM((_B,), jnp.int32),
            pltpu.VMEM((_D, _CW), jnp.float32),
            pltpu.VMEM((_L, 128), jnp.float32),
            pltpu.VMEM((_L, 129), jnp.float32),
            pltpu.VMEM((_L, 128), jnp.float32),
            pltpu.VMEM((_L,), jnp.int32),
            pltpu.VMEM((48,), jnp.float32),
            pltpu.SemaphoreType.DMA,
            pltpu.SemaphoreType.DMA,
        ],
    )
    def kern(uidx_hbm, ut_hbm, iscr_hbm, wb_hbm, oscr_hbm,
             idx_v, hits_v, clist_v, chunk_v, ig_v, igp_v, og_v, kg_v,
             wb_v, sem, sem2):
        wid = lax.axis_index("s") * _NC + lax.axis_index("c")
        pltpu.sync_copy(uidx_hbm, idx_v)
        pltpu.sync_copy(wb_hbm, wb_v)
        n = _scan_bin(idx_v, hits_v, wid, _B // _L, 25)
        lane = lax.broadcasted_iota(jnp.int32, (_L,), 0)
        w_lo = wb_v[pl.ds(0, _L)]
        w_hi = wb_v[pl.ds(_L, _L)]
        b_vec = wb_v[pl.ds(2 * _L, _L)]
        b_s = b_vec[0]

        def slot_body(t, carry):
            c = wid + 32 * t

            @pl.when(c < nchunks)
            def _():
                base = pl.multiple_of(c * _CW, 128)

                @pl.when(c < nchunks - 1)
                def _():
                    pltpu.sync_copy(ut_hbm.at[:, pl.ds(base, _CW)], chunk_v)

                @pl.when(c == nchunks - 1)
                def _():
                    pltpu.sync_copy(ut_hbm.at[:, pl.ds(base, 640)],
                                    chunk_v.at[:, pl.ds(0, 640)])

                m = _rescan(hits_v, n, t, clist_v)

                def grp_body(g):
                    k, off = _group_ids(clist_v, g, m)
                    kg_v[pl.ds(0, _L)] = k
                    pltpu.async_copy(iscr_hbm.at[kg_v],
                                     igp_v.at[:, pl.ds(0, 128)], sem).wait()
                    acc = jnp.zeros((_L,), jnp.float32)
                    for d in range(_D):
                        dv = jnp.full((_L,), d, jnp.int32)
                        uval = plsc.load_gather(chunk_v, [dv, off])
                        ival = plsc.load_gather(igp_v, [lane, dv])
                        wd = (w_lo[d] if d < _L else w_hi[d - _L])
                        acc = acc + uval * ival * wd
                    y = 1.0 / (1.0 + jnp.exp(-(acc + b_s)))
                    zv = jnp.zeros((_L,), jnp.int32)
                    plsc.store_scatter(og_v, [lane, zv], y)
                    pltpu.async_copy(og_v, oscr_hbm.at[kg_v], sem2).wait()
                    return g + _L

                lax.while_loop(lambda g: g < m, grp_body, 0)

            return carry

        lax.fori_loop(0, slots, slot_body, 0)
        return None

    return kern


def kernel(user_indices, item_indices, user_table, item_table, W, b):
    uidx = user_indices.astype(jnp.int32)
    iidx = item_indices.astype(jnp.int32)
    wb = jnp.concatenate([
        W.reshape(-1).astype(jnp.float32),
        jnp.broadcast_to(b.reshape(-1).astype(jnp.float32), (_L,)),
    ])
    iscr = _build_item()(iidx, item_table.T)
    oscr = _build_user()(uidx, user_table.T, iscr, wb)
    return oscr[:_B, 0].reshape(-1, 1)


# speedup vs baseline: 5.4561x; 1.0812x over previous
import functools

import jax
import jax.numpy as jnp
from jax import lax
from jax.experimental import pallas as pl
from jax.experimental.pallas import tpu as pltpu
from jax.experimental.pallas import tpu_sc as plsc

_NC = 2
_NS = 16
_NW = _NC * _NS
_L = 16
_D = 32
_B = 16384
_CW = 1024
_SCR_ROWS = _B + 64
_SINK = _B

_CPT = pltpu.CompilerParams(use_tc_tiling_on_sc=True, needs_layout_passes=False)


def _scan_bin(idx_v, hits_v, wid):
    lane = lax.broadcasted_iota(jnp.int32, (_L,), 0)

    def step(j, ptr):
        v = idx_v[pl.ds(j * _L, _L)]
        cid = lax.shift_right_logical(v, 10)
        mine = (cid & 31) == wid
        m32 = jnp.where(mine, 1, 0)
        ranks = plsc.cumsum(m32)
        tot = ranks[_L - 1]
        k = j * _L + lane
        t = lax.shift_right_logical(cid, 5)
        packed = lax.shift_left(t, 25) | lax.shift_left(k, 10) | (v & 1023)
        plsc.store_scatter(hits_v, [ptr + ranks - 1], packed, mask=mine)
        return ptr + tot

    return lax.fori_loop(0, _B // _L, step, 0)


def _rescan(hits_v, n, t, clist_v):
    nsteps = (n + _L - 1) // _L
    lane = lax.broadcasted_iota(jnp.int32, (_L,), 0)

    def step(j, ptr):
        v = hits_v[pl.ds(j * _L, _L)]
        valid = (j * _L + lane) < n
        mine = jnp.logical_and(valid, lax.shift_right_logical(v, 25) == t)
        m32 = jnp.where(mine, 1, 0)
        ranks = plsc.cumsum(m32)
        tot = ranks[_L - 1]
        plsc.store_scatter(clist_v, [ptr + ranks - 1], v, mask=mine)
        return ptr + tot

    return lax.while_loop(
        lambda c: c[0] < nsteps,
        lambda c: (c[0] + 1, step(c[0], c[1])),
        (0, 0),
    )[1]


def _group_ids(clist_v, g, m, extra):
    lane = lax.broadcasted_iota(jnp.int32, (_L,), 0)
    v = clist_v[pl.ds(g, _L)]
    valid = (g + lane) < m
    k = jnp.where(valid, lax.shift_right_logical(v, 10) & 0x7FFF,
                  _SINK + lane)
    off = jnp.where(valid, (v & 1023) + extra, 0)
    return k, off


def _chunk_base(c, nchunks, vcols):
    pad_end = -(-vcols // 128) * 128
    last = jnp.int32(pad_end - _CW)
    base = jnp.where(c == nchunks - 1, last, c * _CW)
    return pl.multiple_of(base, 128)


def _sweep(tbl_hbm, nchunks, vcols, wid, hits_v, n, clist_v,
           bufs, sems, process):
    extra_last = jnp.int32((nchunks - 1) * _CW - (-(-vcols // 128) * 128 - _CW))

    def fire(c, buf, sem):
        pltpu.async_copy(
            tbl_hbm.at[:, pl.ds(_chunk_base(c, nchunks, vcols), _CW)],
            buf, sem)

    def drain(c, buf, sem):
        pltpu.make_async_copy(
            tbl_hbm.at[:, pl.ds(_chunk_base(c, nchunks, vcols), _CW)],
            buf, sem).wait()

    def handle(c, buf, sem):
        drain(c, buf, sem)
        m = _rescan(hits_v, n, t=lax.shift_right_logical(c, 5), clist_v=clist_v)
        extra = jnp.where(c == nchunks - 1, extra_last, 0)
        process(c, buf, m, extra)

    @pl.when(wid < nchunks)
    def _():
        fire(wid, bufs[0], sems[0])

    def pair_body(tp, carry):
        c0 = wid + 32 * (2 * tp)
        c1 = wid + 32 * (2 * tp + 1)
        c0n = wid + 32 * (2 * tp + 2)

        @pl.when(c1 < nchunks)
        def _():
            fire(c1, bufs[1], sems[1])

        @pl.when(c0 < nchunks)
        def _():
            handle(c0, bufs[0], sems[0])

        @pl.when(c0n < nchunks)
        def _():
            fire(c0n, bufs[0], sems[0])

        @pl.when(c1 < nchunks)
        def _():
            handle(c1, bufs[1], sems[1])

        return carry

    npairs = (nchunks // _NW + 2) // 2 + 1
    lax.fori_loop(0, npairs, pair_body, 0)


@functools.cache
def _build_item():
    mesh = plsc.VectorSubcoreMesh(core_axis_name="c", subcore_axis_name="s")
    nchunks = 98

    @functools.partial(
        pl.kernel,
        mesh=mesh,
        out_type=jax.ShapeDtypeStruct((_SCR_ROWS, 128), jnp.float32),
        compiler_params=_CPT,
        scratch_types=[
            pltpu.VMEM((_B,), jnp.int32),
            pltpu.VMEM((_B,), jnp.int32),
            pltpu.VMEM((_B,), jnp.int32),
            pltpu.VMEM((_D, _CW), jnp.float32),
            pltpu.VMEM((_D, _CW), jnp.float32),
            pltpu.VMEM((_L, 129), jnp.float32),
            pltpu.VMEM((_L,), jnp.int32),
            pltpu.SemaphoreType.DMA,
            pltpu.SemaphoreType.DMA,
            pltpu.SemaphoreType.DMA,
        ],
    )
    def kern(iidx_hbm, it_hbm, iscr_hbm,
             idx_v, hits_v, clist_v, chunk0_v, chunk1_v, gp_v, kg_v,
             sem0, sem1, sems):
        wid = lax.axis_index("s") * _NC + lax.axis_index("c")
        pltpu.sync_copy(iidx_hbm, idx_v)
        n = _scan_bin(idx_v, hits_v, wid)
        lane = lax.broadcasted_iota(jnp.int32, (_L,), 0)

        def process(c, buf, m, extra):
            def grp_body(g):
                k, off = _group_ids(clist_v, g, m, extra)
                kg_v[pl.ds(0, _L)] = k
                for d in range(_D):
                    dv = jnp.full((_L,), d, jnp.int32)
                    val = plsc.load_gather(buf, [dv, off])
                    plsc.store_scatter(gp_v, [lane, dv], val)
                pltpu.async_copy(gp_v.at[:, pl.ds(0, 128)],
                                 iscr_hbm.at[kg_v], sems).wait()
                return g + _L

            lax.while_loop(lambda g: g < m, grp_body, 0)

        _sweep(it_hbm, nchunks, 100000, wid, hits_v, n, clist_v,
               (chunk0_v, chunk1_v), (sem0, sem1), process)
        return None

    return kern


@functools.cache
def _build_user():
    mesh = plsc.VectorSubcoreMesh(core_axis_name="c", subcore_axis_name="s")
    nchunks = 977

    @functools.partial(
        pl.kernel,
        mesh=mesh,
        out_type=jax.ShapeDtypeStruct((_SCR_ROWS, 128), jnp.float32),
        compiler_params=_CPT,
        scratch_types=[
            pltpu.VMEM((_B,), jnp.int32),
            pltpu.VMEM((_B,), jnp.int32),
            pltpu.VMEM((_B,), jnp.int32),
            pltpu.VMEM((_D, _CW), jnp.float32),
            pltpu.VMEM((_D, _CW), jnp.float32),
            pltpu.VMEM((_L, 129), jnp.float32),
            pltpu.VMEM((_L, 128), jnp.float32),
            pltpu.VMEM((_L,), jnp.int32),
            pltpu.VMEM((48,), jnp.float32),
            pltpu.SemaphoreType.DMA,
            pltpu.SemaphoreType.DMA,
            pltpu.SemaphoreType.DMA,
            pltpu.SemaphoreType.DMA,
        ],
    )
    def kern(uidx_hbm, ut_hbm, iscr_hbm, wb_hbm, oscr_hbm,
             idx_v, hits_v, clist_v, chunk0_v, chunk1_v, igp_v, og_v, kg_v,
             wb_v, sem0, sem1, semg, semo):
        wid = lax.axis_index("s") * _NC + lax.axis_index("c")
        pltpu.sync_copy(uidx_hbm, idx_v)
        pltpu.sync_copy(wb_hbm, wb_v)
        n = _scan_bin(idx_v, hits_v, wid)
        lane = lax.broadcasted_iota(jnp.int32, (_L,), 0)
        w_lo = wb_v[pl.ds(0, _L)]
        w_hi = wb_v[pl.ds(_L, _L)]
        b_s = wb_v[pl.ds(2 * _L, _L)][0]
        zv = jnp.zeros((_L,), jnp.int32)

        def process(c, buf, m, extra):
            def grp_body(g):
                k, off = _group_ids(clist_v, g, m, extra)
                kg_v[pl.ds(0, _L)] = k
                pltpu.async_copy(iscr_hbm.at[kg_v],
                                 igp_v.at[:, pl.ds(0, 128)], semg).wait()
                acc = jnp.zeros((_L,), jnp.float32)
                for d in range(_D):
                    dv = jnp.full((_L,), d, jnp.int32)
                    uval = plsc.load_gather(buf, [dv, off])
                    ival = plsc.load_gather(igp_v, [lane, dv])
                    wd = (w_lo[d] if d < _L else w_hi[d - _L])
                    acc = acc + uval * ival * wd
                y = 1.0 / (1.0 + jnp.exp(-(acc + b_s)))
                plsc.store_scatter(og_v, [lane, zv], y)
                pltpu.async_copy(og_v, oscr_hbm.at[kg_v], semo).wait()
                return g + _L

            lax.while_loop(lambda g: g < m, grp_body, 0)

        _sweep(ut_hbm, nchunks, 1000000, wid, hits_v, n, clist_v,
               (chunk0_v, chunk1_v), (sem0, sem1), process)
        return None

    return kern


def kernel(user_indices, item_indices, user_table, item_table, W, b):
    uidx = user_indices.astype(jnp.int32)
    iidx = item_indices.astype(jnp.int32)
    wb = jnp.concatenate([
        W.reshape(-1).astype(jnp.float32),
        jnp.broadcast_to(b.reshape(-1).astype(jnp.float32), (_L,)),
    ])
    iscr = _build_item()(iidx, item_table.T)
    oscr = _build_user()(uidx, user_table.T, iscr, wb)
    return oscr[:_B, 0].reshape(-1, 1)


# speedup vs baseline: 5.5776x; 1.0223x over previous
import functools

import jax
import jax.numpy as jnp
from jax import lax
from jax.experimental import pallas as pl
from jax.experimental.pallas import tpu as pltpu
from jax.experimental.pallas import tpu_sc as plsc

_NC = 2
_NS = 16
_NW = _NC * _NS
_L = 16
_D = 32
_B = 16384
_CW = 1024
_SCR_ROWS = _B + 64
_SINK = _B

_CPT = pltpu.CompilerParams(use_tc_tiling_on_sc=True, needs_layout_passes=False)


def _scan_bin(idx_v, hits_v, wid):
    lane = lax.broadcasted_iota(jnp.int32, (_L,), 0)

    def half(j, v):
        cid = lax.shift_right_logical(v, 10)
        mine = (cid & 31) == wid
        ranks = plsc.cumsum(jnp.where(mine, 1, 0))
        k = j * _L + lane
        t = lax.shift_right_logical(cid, 5)
        packed = lax.shift_left(t, 25) | lax.shift_left(k, 10) | (v & 1023)
        return mine, ranks, ranks[_L - 1], packed

    def step(j, ptr):
        m0, r0, t0, p0 = half(2 * j, idx_v[pl.ds(j * 2 * _L, _L)])
        m1, r1, t1, p1 = half(2 * j + 1, idx_v[pl.ds(j * 2 * _L + _L, _L)])
        plsc.store_scatter(hits_v, [ptr + r0 - 1], p0, mask=m0)
        plsc.store_scatter(hits_v, [ptr + t0 + r1 - 1], p1, mask=m1)
        return ptr + t0 + t1

    return lax.fori_loop(0, _B // (2 * _L), step, 0)


def _rescan(hits_v, n, t, clist_v):
    nsteps = (n + _L - 1) // _L
    lane = lax.broadcasted_iota(jnp.int32, (_L,), 0)

    def step(j, ptr):
        v = hits_v[pl.ds(j * _L, _L)]
        valid = (j * _L + lane) < n
        mine = jnp.logical_and(valid, lax.shift_right_logical(v, 25) == t)
        m32 = jnp.where(mine, 1, 0)
        ranks = plsc.cumsum(m32)
        tot = ranks[_L - 1]
        plsc.store_scatter(clist_v, [ptr + ranks - 1], v, mask=mine)
        return ptr + tot

    return lax.while_loop(
        lambda c: c[0] < nsteps,
        lambda c: (c[0] + 1, step(c[0], c[1])),
        (0, 0),
    )[1]


def _group_ids(clist_v, g, m, extra):
    lane = lax.broadcasted_iota(jnp.int32, (_L,), 0)
    v = clist_v[pl.ds(g, _L)]
    valid = (g + lane) < m
    k = jnp.where(valid, lax.shift_right_logical(v, 10) & 0x7FFF,
                  _SINK + lane)
    off = jnp.where(valid, (v & 1023) + extra, 0)
    return k, off


def _chunk_base(c, nchunks, vcols):
    pad_end = -(-vcols // 128) * 128
    last = jnp.int32(pad_end - _CW)
    base = jnp.where(c == nchunks - 1, last, c * _CW)
    return pl.multiple_of(base, 128)


def _sweep(tbl_hbm, nchunks, vcols, wid, hits_v, scan, clist_v,
           bufs, sems, process):
    extra_last = jnp.int32((nchunks - 1) * _CW - (-(-vcols // 128) * 128 - _CW))

    def fire(c, buf, sem):
        pltpu.async_copy(
            tbl_hbm.at[:, pl.ds(_chunk_base(c, nchunks, vcols), _CW)],
            buf, sem)

    def drain(c, buf, sem):
        pltpu.make_async_copy(
            tbl_hbm.at[:, pl.ds(_chunk_base(c, nchunks, vcols), _CW)],
            buf, sem).wait()

    @pl.when(wid < nchunks)
    def _():
        fire(wid, bufs[0], sems[0])

    n = scan()

    def handle(c, buf, sem):
        drain(c, buf, sem)
        m = _rescan(hits_v, n, t=lax.shift_right_logical(c, 5), clist_v=clist_v)
        extra = jnp.where(c == nchunks - 1, extra_last, 0)
        process(c, buf, m, extra)

    def pair_body(tp, carry):
        c0 = wid + 32 * (2 * tp)
        c1 = wid + 32 * (2 * tp + 1)
        c0n = wid + 32 * (2 * tp + 2)

        @pl.when(c1 < nchunks)
        def _():
            fire(c1, bufs[1], sems[1])

        @pl.when(c0 < nchunks)
        def _():
            handle(c0, bufs[0], sems[0])

        @pl.when(c0n < nchunks)
        def _():
            fire(c0n, bufs[0], sems[0])

        @pl.when(c1 < nchunks)
        def _():
            handle(c1, bufs[1], sems[1])

        return carry

    npairs = (nchunks // _NW + 2) // 2 + 1
    lax.fori_loop(0, npairs, pair_body, 0)


@functools.cache
def _build_item():
    mesh = plsc.VectorSubcoreMesh(core_axis_name="c", subcore_axis_name="s")
    nchunks = 98

    @functools.partial(
        pl.kernel,
        mesh=mesh,
        out_type=jax.ShapeDtypeStruct((_SCR_ROWS, 128), jnp.float32),
        compiler_params=_CPT,
        scratch_types=[
            pltpu.VMEM((_B,), jnp.int32),
            pltpu.VMEM((_B,), jnp.int32),
            pltpu.VMEM((_B,), jnp.int32),
            pltpu.VMEM((_D, _CW), jnp.float32),
            pltpu.VMEM((_D, _CW), jnp.float32),
            pltpu.VMEM((_L, 129), jnp.float32),
            pltpu.VMEM((_L,), jnp.int32),
            pltpu.SemaphoreType.DMA,
            pltpu.SemaphoreType.DMA,
            pltpu.SemaphoreType.DMA,
        ],
    )
    def kern(iidx_hbm, it_hbm, iscr_hbm,
             idx_v, hits_v, clist_v, chunk0_v, chunk1_v, gp_v, kg_v,
             sem0, sem1, sems):
        wid = lax.axis_index("s") * _NC + lax.axis_index("c")
        pltpu.sync_copy(iidx_hbm, idx_v)
        lane = lax.broadcasted_iota(jnp.int32, (_L,), 0)

        def process(c, buf, m, extra):
            def grp_body(g):
                k, off = _group_ids(clist_v, g, m, extra)
                kg_v[pl.ds(0, _L)] = k
                for d in range(_D):
                    dv = jnp.full((_L,), d, jnp.int32)
                    val = plsc.load_gather(buf, [dv, off])
                    plsc.store_scatter(gp_v, [lane, dv], val)
                pltpu.async_copy(gp_v.at[:, pl.ds(0, 128)],
                                 iscr_hbm.at[kg_v], sems).wait()
                return g + _L

            lax.while_loop(lambda g: g < m, grp_body, 0)

        _sweep(it_hbm, nchunks, 100000, wid, hits_v,
               lambda: _scan_bin(idx_v, hits_v, wid), clist_v,
               (chunk0_v, chunk1_v), (sem0, sem1), process)
        return None

    return kern


@functools.cache
def _build_user():
    mesh = plsc.VectorSubcoreMesh(core_axis_name="c", subcore_axis_name="s")
    nchunks = 977

    @functools.partial(
        pl.kernel,
        mesh=mesh,
        out_type=jax.ShapeDtypeStruct((_SCR_ROWS, 128), jnp.float32),
        compiler_params=_CPT,
        scratch_types=[
            pltpu.VMEM((_B,), jnp.int32),
            pltpu.VMEM((_B,), jnp.int32),
            pltpu.VMEM((_B,), jnp.int32),
            pltpu.VMEM((_D, _CW), jnp.float32),
            pltpu.VMEM((_D, _CW), jnp.float32),
            pltpu.VMEM((_L, 129), jnp.float32),
            pltpu.VMEM((_L, 128), jnp.float32),
            pltpu.VMEM((_L,), jnp.int32),
            pltpu.VMEM((48,), jnp.float32),
            pltpu.SemaphoreType.DMA,
            pltpu.SemaphoreType.DMA,
            pltpu.SemaphoreType.DMA,
            pltpu.SemaphoreType.DMA,
        ],
    )
    def kern(uidx_hbm, ut_hbm, iscr_hbm, wb_hbm, oscr_hbm,
             idx_v, hits_v, clist_v, chunk0_v, chunk1_v, igp_v, og_v, kg_v,
             wb_v, sem0, sem1, semg, semo):
        wid = lax.axis_index("s") * _NC + lax.axis_index("c")
        pltpu.sync_copy(uidx_hbm, idx_v)
        pltpu.sync_copy(wb_hbm, wb_v)
        lane = lax.broadcasted_iota(jnp.int32, (_L,), 0)
        w_lo = wb_v[pl.ds(0, _L)]
        w_hi = wb_v[pl.ds(_L, _L)]
        b_s = wb_v[pl.ds(2 * _L, _L)][0]
        zv = jnp.zeros((_L,), jnp.int32)

        def process(c, buf, m, extra):
            def grp_body(g):
                k, off = _group_ids(clist_v, g, m, extra)
                kg_v[pl.ds(0, _L)] = k
                pltpu.async_copy(iscr_hbm.at[kg_v],
                                 igp_v.at[:, pl.ds(0, 128)], semg).wait()
                acc = jnp.zeros((_L,), jnp.float32)
                for d in range(_D):
                    dv = jnp.full((_L,), d, jnp.int32)
                    uval = plsc.load_gather(buf, [dv, off])
                    ival = plsc.load_gather(igp_v, [lane, dv])
                    wd = (w_lo[d] if d < _L else w_hi[d - _L])
                    acc = acc + uval * ival * wd
                y = 1.0 / (1.0 + jnp.exp(-(acc + b_s)))
                plsc.store_scatter(og_v, [lane, zv], y)
                pltpu.async_copy(og_v, oscr_hbm.at[kg_v], semo).wait()
                return g + _L

            lax.while_loop(lambda g: g < m, grp_body, 0)

        _sweep(ut_hbm, nchunks, 1000000, wid, hits_v,
               lambda: _scan_bin(idx_v, hits_v, wid), clist_v,
               (chunk0_v, chunk1_v), (sem0, sem1), process)
        return None

    return kern


def kernel(user_indices, item_indices, user_table, item_table, W, b):
    uidx = user_indices.astype(jnp.int32)
    iidx = item_indices.astype(jnp.int32)
    wb = jnp.concatenate([
        W.reshape(-1).astype(jnp.float32),
        jnp.broadcast_to(b.reshape(-1).astype(jnp.float32), (_L,)),
    ])
    iscr = _build_item()(iidx, item_table.T)
    oscr = _build_user()(uidx, user_table.T, iscr, wb)
    return oscr[:_B, 0].reshape(-1, 1)


# speedup vs baseline: 5.6439x; 1.0119x over previous
import functools

import jax
import jax.numpy as jnp
from jax import lax
from jax.experimental import pallas as pl
from jax.experimental.pallas import tpu as pltpu
from jax.experimental.pallas import tpu_sc as plsc

_NC = 2
_NS = 16
_NW = _NC * _NS
_L = 16
_D = 32
_B = 16384
_CW = 1024
_SCR_ROWS = _B + 64
_SINK = _B

_CPT = pltpu.CompilerParams(use_tc_tiling_on_sc=True, needs_layout_passes=False)


def _scan_bin(idx_v, hits_v, wid):
    lane = lax.broadcasted_iota(jnp.int32, (_L,), 0)

    def half(j, v):
        cid = lax.shift_right_logical(v, 10)
        mine = (cid & 31) == wid
        ranks = plsc.cumsum(jnp.where(mine, 1, 0))
        k = j * _L + lane
        t = lax.shift_right_logical(cid, 5)
        packed = lax.shift_left(t, 25) | lax.shift_left(k, 10) | (v & 1023)
        return mine, ranks, ranks[_L - 1], packed

    def step(j, ptr):
        m0, r0, t0, p0 = half(2 * j, idx_v[pl.ds(j * 2 * _L, _L)])
        m1, r1, t1, p1 = half(2 * j + 1, idx_v[pl.ds(j * 2 * _L + _L, _L)])
        plsc.store_scatter(hits_v, [ptr + r0 - 1], p0, mask=m0)
        plsc.store_scatter(hits_v, [ptr + t0 + r1 - 1], p1, mask=m1)
        return ptr + t0 + t1

    return lax.fori_loop(0, _B // (2 * _L), step, 0)


def _rescan(hits_v, n, t, clist_v):
    nsteps = (n + _L - 1) // _L
    lane = lax.broadcasted_iota(jnp.int32, (_L,), 0)

    def step(j, ptr):
        v = hits_v[pl.ds(j * _L, _L)]
        valid = (j * _L + lane) < n
        mine = jnp.logical_and(valid, lax.shift_right_logical(v, 25) == t)
        m32 = jnp.where(mine, 1, 0)
        ranks = plsc.cumsum(m32)
        tot = ranks[_L - 1]
        plsc.store_scatter(clist_v, [ptr + ranks - 1], v, mask=mine)
        return ptr + tot

    return lax.while_loop(
        lambda c: c[0] < nsteps,
        lambda c: (c[0] + 1, step(c[0], c[1])),
        (0, 0),
    )[1]


def _group_ids(clist_v, g, m, extra):
    lane = lax.broadcasted_iota(jnp.int32, (_L,), 0)
    v = clist_v[pl.ds(g, _L)]
    valid = (g + lane) < m
    k = jnp.where(valid, lax.shift_right_logical(v, 10) & 0x7FFF,
                  _SINK + lane)
    off = jnp.where(valid, (v & 1023) + extra, 0)
    return k, off


def _chunk_base(c, nchunks, vcols):
    pad_end = -(-vcols // 128) * 128
    last = jnp.int32(pad_end - _CW)
    base = jnp.where(c == nchunks - 1, last, c * _CW)
    return pl.multiple_of(base, 128)


def _sweep(tbl_hbm, nchunks, vcols, wid, hits_v, scan, clist_v,
           bufs, sems, process):
    extra_last = jnp.int32((nchunks - 1) * _CW - (-(-vcols // 128) * 128 - _CW))

    def fire(c, buf, sem):
        pltpu.async_copy(
            tbl_hbm.at[:, pl.ds(_chunk_base(c, nchunks, vcols), _CW)],
            buf, sem)

    def drain(c, buf, sem):
        pltpu.make_async_copy(
            tbl_hbm.at[:, pl.ds(_chunk_base(c, nchunks, vcols), _CW)],
            buf, sem).wait()

    @pl.when(wid < nchunks)
    def _():
        fire(wid, bufs[0], sems[0])

    n = scan()

    def handle(c, buf, sem):
        drain(c, buf, sem)
        m = _rescan(hits_v, n, t=lax.shift_right_logical(c, 5), clist_v=clist_v)
        extra = jnp.where(c == nchunks - 1, extra_last, 0)
        process(c, buf, m, extra)

    def pair_body(tp, carry):
        c0 = wid + 32 * (2 * tp)
        c1 = wid + 32 * (2 * tp + 1)
        c0n = wid + 32 * (2 * tp + 2)

        @pl.when(c1 < nchunks)
        def _():
            fire(c1, bufs[1], sems[1])

        @pl.when(c0 < nchunks)
        def _():
            handle(c0, bufs[0], sems[0])

        @pl.when(c0n < nchunks)
        def _():
            fire(c0n, bufs[0], sems[0])

        @pl.when(c1 < nchunks)
        def _():
            handle(c1, bufs[1], sems[1])

        return carry

    npairs = (nchunks // _NW + 2) // 2 + 1
    lax.fori_loop(0, npairs, pair_body, 0)


@functools.cache
def _build_item():
    mesh = plsc.VectorSubcoreMesh(core_axis_name="c", subcore_axis_name="s")
    nchunks = 98

    @functools.partial(
        pl.kernel,
        mesh=mesh,
        out_type=jax.ShapeDtypeStruct((_SCR_ROWS, 128), jnp.float32),
        compiler_params=_CPT,
        scratch_types=[
            pltpu.VMEM((_B,), jnp.int32),
            pltpu.VMEM((_B,), jnp.int32),
            pltpu.VMEM((_B,), jnp.int32),
            pltpu.VMEM((_D, _CW), jnp.float32),
            pltpu.VMEM((_D, _CW), jnp.float32),
            pltpu.VMEM((_L, 129), jnp.float32),
            pltpu.VMEM((_L, 129), jnp.float32),
            pltpu.VMEM((_L,), jnp.int32),
            pltpu.VMEM((_L,), jnp.int32),
            pltpu.SemaphoreType.DMA,
            pltpu.SemaphoreType.DMA,
            pltpu.SemaphoreType.DMA,
            pltpu.SemaphoreType.DMA,
        ],
    )
    def kern(iidx_hbm, it_hbm, iscr_hbm,
             idx_v, hits_v, clist_v, chunk0_v, chunk1_v, gp_v, gp1_v,
             kg_v, kg1_v, sem0, sem1, sems, sems1):
        wid = lax.axis_index("s") * _NC + lax.axis_index("c")
        pltpu.sync_copy(iidx_hbm, idx_v)
        lane = lax.broadcasted_iota(jnp.int32, (_L,), 0)

        def process(c, buf, m, extra):
            def one(g, gp, kg, sem):
                @pl.when(g >= 2 * _L)
                def _():
                    pltpu.make_async_copy(gp.at[:, pl.ds(0, 128)],
                                          iscr_hbm.at[kg], sem).wait()
                k, off = _group_ids(clist_v, g, m, extra)
                kg[pl.ds(0, _L)] = k
                for d in range(_D):
                    dv = jnp.full((_L,), d, jnp.int32)
                    val = plsc.load_gather(buf, [dv, off])
                    plsc.store_scatter(gp, [lane, dv], val)
                pltpu.async_copy(gp.at[:, pl.ds(0, 128)],
                                 iscr_hbm.at[kg], sem)

            def grp_body(g):
                par = lax.shift_right_logical(g, 4) & 1

                @pl.when(par == 0)
                def _():
                    one(g, gp_v, kg_v, sems)

                @pl.when(par == 1)
                def _():
                    one(g, gp1_v, kg1_v, sems1)

                return g + _L

            lax.while_loop(lambda g: g < m, grp_body, 0)

            @pl.when(m > 0)
            def _():
                pltpu.make_async_copy(gp_v.at[:, pl.ds(0, 128)],
                                      iscr_hbm.at[kg_v], sems).wait()

            @pl.when(m > _L)
            def _():
                pltpu.make_async_copy(gp1_v.at[:, pl.ds(0, 128)],
                                      iscr_hbm.at[kg1_v], sems1).wait()

        _sweep(it_hbm, nchunks, 100000, wid, hits_v,
               lambda: _scan_bin(idx_v, hits_v, wid), clist_v,
               (chunk0_v, chunk1_v), (sem0, sem1), process)
        return None

    return kern


@functools.cache
def _build_user():
    mesh = plsc.VectorSubcoreMesh(core_axis_name="c", subcore_axis_name="s")
    nchunks = 977

    @functools.partial(
        pl.kernel,
        mesh=mesh,
        out_type=jax.ShapeDtypeStruct((_SCR_ROWS, 128), jnp.float32),
        compiler_params=_CPT,
        scratch_types=[
            pltpu.VMEM((_B,), jnp.int32),
            pltpu.VMEM((_B,), jnp.int32),
            pltpu.VMEM((_B,), jnp.int32),
            pltpu.VMEM((_D, _CW), jnp.float32),
            pltpu.VMEM((_D, _CW), jnp.float32),
            pltpu.VMEM((_L, 129), jnp.float32),
            pltpu.VMEM((_L, 128), jnp.float32),
            pltpu.VMEM((_L, 128), jnp.float32),
            pltpu.VMEM((_L,), jnp.int32),
            pltpu.VMEM((_L,), jnp.int32),
            pltpu.VMEM((48,), jnp.float32),
            pltpu.SemaphoreType.DMA,
            pltpu.SemaphoreType.DMA,
            pltpu.SemaphoreType.DMA,
            pltpu.SemaphoreType.DMA,
            pltpu.SemaphoreType.DMA,
        ],
    )
    def kern(uidx_hbm, ut_hbm, iscr_hbm, wb_hbm, oscr_hbm,
             idx_v, hits_v, clist_v, chunk0_v, chunk1_v, igp_v, og_v, og1_v,
             kg_v, kg1_v, wb_v, sem0, sem1, semg, semo, semo1):
        wid = lax.axis_index("s") * _NC + lax.axis_index("c")
        pltpu.sync_copy(uidx_hbm, idx_v)
        pltpu.sync_copy(wb_hbm, wb_v)
        lane = lax.broadcasted_iota(jnp.int32, (_L,), 0)
        w_lo = wb_v[pl.ds(0, _L)]
        w_hi = wb_v[pl.ds(_L, _L)]
        b_s = wb_v[pl.ds(2 * _L, _L)][0]
        zv = jnp.zeros((_L,), jnp.int32)

        def process(c, buf, m, extra):
            def one(g, og, kg, sem):
                @pl.when(g >= 2 * _L)
                def _():
                    pltpu.make_async_copy(og, oscr_hbm.at[kg], sem).wait()
                k, off = _group_ids(clist_v, g, m, extra)
                kg[pl.ds(0, _L)] = k
                pltpu.async_copy(iscr_hbm.at[kg],
                                 igp_v.at[:, pl.ds(0, 128)], semg).wait()
                acc = jnp.zeros((_L,), jnp.float32)
                for d in range(_D):
                    dv = jnp.full((_L,), d, jnp.int32)
                    uval = plsc.load_gather(buf, [dv, off])
                    ival = plsc.load_gather(igp_v, [lane, dv])
                    wd = (w_lo[d] if d < _L else w_hi[d - _L])
                    acc = acc + uval * ival * wd
                y = 1.0 / (1.0 + jnp.exp(-(acc + b_s)))
                plsc.store_scatter(og, [lane, zv], y)
                pltpu.async_copy(og, oscr_hbm.at[kg], sem)

            def grp_body(g):
                par = lax.shift_right_logical(g, 4) & 1

                @pl.when(par == 0)
                def _():
                    one(g, og_v, kg_v, semo)

                @pl.when(par == 1)
                def _():
                    one(g, og1_v, kg1_v, semo1)

                return g + _L

            lax.while_loop(lambda g: g < m, grp_body, 0)

            @pl.when(m > 0)
            def _():
                pltpu.make_async_copy(og_v, oscr_hbm.at[kg_v], semo).wait()

            @pl.when(m > _L)
            def _():
                pltpu.make_async_copy(og1_v, oscr_hbm.at[kg1_v], semo1).wait()

        _sweep(ut_hbm, nchunks, 1000000, wid, hits_v,
               lambda: _scan_bin(idx_v, hits_v, wid), clist_v,
               (chunk0_v, chunk1_v), (sem0, sem1), process)
        return None

    return kern


def kernel(user_indices, item_indices, user_table, item_table, W, b):
    uidx = user_indices.astype(jnp.int32)
    iidx = item_indices.astype(jnp.int32)
    wb = jnp.concatenate([
        W.reshape(-1).astype(jnp.float32),
        jnp.broadcast_to(b.reshape(-1).astype(jnp.float32), (_L,)),
    ])
    iscr = _build_item()(iidx, item_table.T)
    oscr = _build_user()(uidx, user_table.T, iscr, wb)
    return oscr[:_B, 0].reshape(-1, 1)
